# baseline (speedup 1.0000x reference)
import functools

import jax
import jax.numpy as jnp
from jax import lax
from jax.experimental import pallas as pl
from jax.experimental.pallas import tpu as pltpu
from jax.experimental.pallas import tpu_sc as plsc

EMBED_DIM = 64
D_MODEL = 2 * EMBED_DIM
D_INNER = 2 * D_MODEL
HID = 2 * D_MODEL
STEPS = 4
FEW = 5
SUP_PAD = 8

_NC, _NS = 2, 16
_NW = _NC * _NS


@functools.lru_cache(maxsize=None)
def _make_sc_gather(n_total: int, n_per_w: int):
    mesh = plsc.VectorSubcoreMesh(core_axis_name="c", subcore_axis_name="s")

    @functools.partial(
        pl.kernel,
        mesh=mesh,
        compiler_params=pltpu.CompilerParams(use_tc_tiling_on_sc=False),
        out_type=jax.ShapeDtypeStruct((n_total, EMBED_DIM), jnp.float32),
        scratch_types=[
            pltpu.VMEM((n_per_w,), jnp.int32),
            pltpu.VMEM((n_per_w, EMBED_DIM), jnp.float32),
            pltpu.SemaphoreType.DMA,
        ],
    )
    def gather_k(table_hbm, idx_hbm, out_hbm, idx_v, rows_v, sem):
        wid = lax.axis_index("s") * _NC + lax.axis_index("c")
        base = wid * n_per_w
        pltpu.sync_copy(idx_hbm.at[pl.ds(base, n_per_w)], idx_v)
        pltpu.async_copy(table_hbm.at[idx_v], rows_v, sem).wait()
        pltpu.sync_copy(rows_v, out_hbm.at[pl.ds(base, n_per_w)])

    return gather_k


def _dense_body(q_ref, s_ref, W1T_ref, b1_ref, W2T_ref, b2_ref, g_ref,
                bb_ref, WihT_ref, bih_ref, WhhT_ref, bhh_ref, out_ref):
    f32 = jnp.float32
    s = s_ref[...]
    h1 = jnp.maximum(
        jnp.dot(s, W1T_ref[...], preferred_element_type=f32) + b1_ref[...], 0.0)
    h2 = jnp.dot(h1, W2T_ref[...], preferred_element_type=f32) + b2_ref[...] + s
    mu = jnp.mean(h2, axis=-1, keepdims=True)
    var = jnp.mean((h2 - mu) ** 2, axis=-1, keepdims=True)
    sg = g_ref[...] * (h2 - mu) / (jnp.sqrt(var) + 1e-6) + bb_ref[...]

    col = lax.broadcasted_iota(jnp.int32, (1, SUP_PAD), 1)
    neg = jnp.where(col < FEW, 0.0, -1e30)

    q = q_ref[...]
    bm = q.shape[0]
    xW = jnp.dot(q, WihT_ref[...], preferred_element_type=f32) + bih_ref[...]
    h_r = jnp.zeros((bm, HID), f32)
    c = jnp.zeros((bm, HID), f32)
    h = q
    for step in range(STEPS):
        gates = xW + bhh_ref[...]
        if step > 0:
            gates = gates + jnp.dot(h_r, WhhT_ref[...],
                                    preferred_element_type=f32)
        sig = lambda x: 0.5 * jnp.tanh(0.5 * x) + 0.5
        i_g = sig(gates[:, 0:HID])
        f_g = sig(gates[:, HID:2 * HID])
        g_g = jnp.tanh(gates[:, 2 * HID:3 * HID])
        o_g = sig(gates[:, 3 * HID:4 * HID])
        c = f_g * c + i_g * g_g
        h_new = o_g * jnp.tanh(c)
        h = q + h_new[:, :D_MODEL]
        logits = lax.dot_general(h, sg, (((1,), (1,)), ((), ())),
                                 preferred_element_type=f32) + neg
        attn = jax.nn.softmax(logits, axis=1)
        r = jnp.dot(attn, sg, preferred_element_type=f32)
        h_r = jnp.concatenate([h, r], axis=1)
    out_ref[...] = lax.dot_general(h, sg, (((1,), (1,)), ((), ())),
                                   preferred_element_type=f32)


def _dense_call(q, s8, W1T, b1, W2T, b2, ln_g, ln_b, WihT, bih, WhhT, bhh,
                bm: int, interpret: bool = False):
    B = q.shape[0]
    grid = (B // bm,)
    full = lambda shape: pl.BlockSpec(shape, lambda i: (0, 0))
    return pl.pallas_call(
        _dense_body,
        grid=grid,
        in_specs=[
            pl.BlockSpec((bm, D_MODEL), lambda i: (i, 0)),
            full((SUP_PAD, D_MODEL)),
            full((D_MODEL, D_INNER)),
            full((1, D_INNER)),
            full((D_INNER, D_MODEL)),
            full((1, D_MODEL)),
            full((1, D_MODEL)),
            full((1, D_MODEL)),
            full((D_MODEL, 4 * HID)),
            full((1, 4 * HID)),
            full((HID, 4 * HID)),
            full((1, 4 * HID)),
        ],
        out_specs=pl.BlockSpec((bm, SUP_PAD), lambda i: (i, 0)),
        out_shape=jax.ShapeDtypeStruct((B, SUP_PAD), jnp.float32),
        compiler_params=pltpu.CompilerParams(
            dimension_semantics=("arbitrary",)),
        interpret=interpret,
    )(q, s8, W1T, b1, W2T, b2, ln_g, ln_b, WihT, bih, WhhT, bhh)


def kernel(query, support, emb, W1, b1, W2, b2, ln_g, ln_b, W_ih, W_hh,
           b_ih, b_hh):
    B = query.shape[0]
    n_q = B * 2
    n_s = FEW * 2
    align = 8 * _NW
    n_total = ((n_q + n_s + align - 1) // align) * align
    idx = jnp.concatenate([
        query.reshape(-1), support.reshape(-1),
        jnp.zeros((n_total - n_q - n_s,), jnp.int32)])
    rows = _make_sc_gather(n_total, n_total // _NW)(emb, idx)
    q = rows[:n_q].reshape(B, D_MODEL)
    s = rows[n_q:n_q + n_s].reshape(FEW, D_MODEL)
    s8 = jnp.concatenate([s, jnp.zeros((SUP_PAD - FEW, D_MODEL),
                                       jnp.float32)], axis=0)
    scores8 = _dense_call(
        q, s8, W1.T, b1[None, :], W2.T, b2[None, :], ln_g[None, :],
        ln_b[None, :], W_ih.T, b_ih[None, :], W_hh.T, b_hh[None, :], bm=2048)
    return scores8[:, :FEW]


# speedup vs baseline: 2.0095x; 2.0095x over previous
import functools

import jax
import jax.numpy as jnp
from jax import lax
from jax.experimental import pallas as pl
from jax.experimental.pallas import tpu as pltpu
from jax.experimental.pallas import tpu_sc as plsc

EMBED_DIM = 64
D_MODEL = 2 * EMBED_DIM
D_INNER = 2 * D_MODEL
HID = 2 * D_MODEL
STEPS = 4
FEW = 5
SUP_PAD = 8

_NC, _NS = 2, 16
_NW = _NC * _NS


@functools.lru_cache(maxsize=None)
def _make_sc_gather(n_total: int, n_per_w: int):
    mesh = plsc.VectorSubcoreMesh(core_axis_name="c", subcore_axis_name="s")

    @functools.partial(
        pl.kernel,
        mesh=mesh,
        compiler_params=pltpu.CompilerParams(use_tc_tiling_on_sc=False),
        out_type=jax.ShapeDtypeStruct((n_total, EMBED_DIM), jnp.float32),
        scratch_types=[
            pltpu.VMEM((n_per_w,), jnp.int32),
            pltpu.VMEM((n_per_w, EMBED_DIM), jnp.float32),
            pltpu.SemaphoreType.DMA,
        ],
    )
    def gather_k(table_hbm, idx_hbm, out_hbm, idx_v, rows_v, sem):
        wid = lax.axis_index("s") * _NC + lax.axis_index("c")
        base = wid * n_per_w
        pltpu.sync_copy(idx_hbm.at[pl.ds(base, n_per_w)], idx_v)
        pltpu.async_copy(table_hbm.at[idx_v], rows_v, sem).wait()
        pltpu.sync_copy(rows_v, out_hbm.at[pl.ds(base, n_per_w)])

    return gather_k


def _dense_body(q_ref, s_ref, W1T_ref, b1_ref, W2T_ref, b2_ref, g_ref,
                bb_ref, WihT_ref, bih_ref, WhhT_ref, bhh_ref, out_ref):
    f32 = jnp.float32
    s = s_ref[...]
    h1 = jnp.maximum(
        jnp.dot(s, W1T_ref[...], preferred_element_type=f32) + b1_ref[...], 0.0)
    h2 = jnp.dot(h1, W2T_ref[...], preferred_element_type=f32) + b2_ref[...] + s
    mu = jnp.mean(h2, axis=-1, keepdims=True)
    var = jnp.mean((h2 - mu) ** 2, axis=-1, keepdims=True)
    sg = g_ref[...] * (h2 - mu) / (jnp.sqrt(var) + 1e-6) + bb_ref[...]

    col = lax.broadcasted_iota(jnp.int32, (1, SUP_PAD), 1)
    neg = jnp.where(col < FEW, 0.0, -1e30)

    q = q_ref[...]
    bm = q.shape[0]
    xW = jnp.dot(q, WihT_ref[...], preferred_element_type=f32) + bih_ref[...]
    h_r = jnp.zeros((bm, HID), f32)
    c = jnp.zeros((bm, HID), f32)
    h = q
    for step in range(STEPS):
        gates = xW + bhh_ref[...]
        if step > 0:
            gates = gates + jnp.dot(h_r, WhhT_ref[...],
                                    preferred_element_type=f32)
        sig = lambda x: 0.5 * jnp.tanh(0.5 * x) + 0.5
        i_g = sig(gates[:, 0:HID])
        f_g = sig(gates[:, HID:2 * HID])
        g_g = jnp.tanh(gates[:, 2 * HID:3 * HID])
        o_g = sig(gates[:, 3 * HID:4 * HID])
        c = f_g * c + i_g * g_g
        h_new = o_g * jnp.tanh(c)
        h = q + h_new[:, :D_MODEL]
        logits = lax.dot_general(h, sg, (((1,), (1,)), ((), ())),
                                 preferred_element_type=f32) + neg
        attn = jax.nn.softmax(logits, axis=1)
        r = jnp.dot(attn, sg, preferred_element_type=f32)
        h_r = jnp.concatenate([h, r], axis=1)
    out_ref[...] = lax.dot_general(h, sg, (((1,), (1,)), ((), ())),
                                   preferred_element_type=f32)


def _dense_call(q, s8, W1T, b1, W2T, b2, ln_g, ln_b, WihT, bih, WhhT, bhh,
                bm: int, interpret: bool = False):
    B = q.shape[0]
    grid = (B // bm,)
    full = lambda shape: pl.BlockSpec(shape, lambda i: (0, 0))
    return pl.pallas_call(
        _dense_body,
        grid=grid,
        in_specs=[
            pl.BlockSpec((bm, D_MODEL), lambda i: (i, 0)),
            full((SUP_PAD, D_MODEL)),
            full((D_MODEL, D_INNER)),
            full((1, D_INNER)),
            full((D_INNER, D_MODEL)),
            full((1, D_MODEL)),
            full((1, D_MODEL)),
            full((1, D_MODEL)),
            full((D_MODEL, 4 * HID)),
            full((1, 4 * HID)),
            full((HID, 4 * HID)),
            full((1, 4 * HID)),
        ],
        out_specs=pl.BlockSpec((bm, SUP_PAD), lambda i: (i, 0)),
        out_shape=jax.ShapeDtypeStruct((B, SUP_PAD), jnp.float32),
        compiler_params=pltpu.CompilerParams(
            dimension_semantics=("arbitrary",)),
        interpret=interpret,
    )(q, s8, W1T, b1, W2T, b2, ln_g, ln_b, WihT, bih, WhhT, bhh)


def kernel(query, support, emb, W1, b1, W2, b2, ln_g, ln_b, W_ih, W_hh,
           b_ih, b_hh):
    B = query.shape[0]
    n_q = B * 2
    n_s = FEW * 2
    align = 8 * _NW
    n_total = ((n_q + n_s + align - 1) // align) * align
    idx = jnp.concatenate([
        query.reshape(-1), support.reshape(-1),
        jnp.zeros((n_total - n_q - n_s,), jnp.int32)])
    rows = jnp.take(emb, idx, axis=0)
    q = rows[:n_q].reshape(B, D_MODEL)
    s = rows[n_q:n_q + n_s].reshape(FEW, D_MODEL)
    s8 = jnp.concatenate([s, jnp.zeros((SUP_PAD - FEW, D_MODEL),
                                       jnp.float32)], axis=0)
    scores8 = _dense_call(
        q, s8, W1.T, b1[None, :], W2.T, b2[None, :], ln_g[None, :],
        ln_b[None, :], W_ih.T, b_ih[None, :], W_hh.T, b_hh[None, :], bm=2048)
    return scores8[:, :FEW]
